# Initial kernel scaffold; baseline (speedup 1.0000x reference)
import jax
import jax.numpy as jnp
from jax import lax
from jax.experimental import pallas as pl
from jax.experimental.pallas import tpu as pltpu
from jax.experimental.pallas import tpu_sc as plsc

N, D = 320000, 128
NSEG = 10000
NC, NS = 2, 16
NW = NC * NS
ROWS_PER_TILE = N // NW
W = 128
NWIN = ROWS_PER_TILE // W
TAIL = ROWS_PER_TILE - NWIN * W
ZROWS = 125
GW = 80
NGWIN = NSEG // GW


def _pool_body(x_hbm, segs_hbm, y0_hbm, y1_hbm,
               acc, buf, ibuf, zbuf, tbx, tbi, sx0, sx1, si0, si1):
    cid = lax.axis_index("c")
    sid = lax.axis_index("s")
    g = cid * NS + sid
    row0 = g * ROWS_PER_TILE

    z = jnp.zeros((16,), jnp.float32)

    def zrow(r, carry):
        for c in range(D // 16):
            zbuf[r, pl.ds(c * 16, 16)] = z
        return carry

    lax.fori_loop(0, ZROWS, zrow, 0)
    for j in range(625 // ZROWS):
        pltpu.sync_copy(zbuf, acc.at[pl.ds(sid * 625 + j * ZROWS, ZROWS)])
    plsc.subcore_barrier()

    sx = (sx0, sx1)
    si = (si0, si1)

    def start(w, b):
        base = row0 + w * W
        pltpu.async_copy(x_hbm.at[pl.ds(base, W)], buf.at[b], sx[b])
        pltpu.async_copy(segs_hbm.at[pl.ds(base, W)], ibuf.at[b], si[b])

    def wait(b):
        pltpu.make_async_copy(x_hbm.at[pl.ds(0, W)], buf.at[b], sx[b]).wait()
        pltpu.make_async_copy(segs_hbm.at[pl.ds(0, W)], ibuf.at[b], si[b]).wait()

    start(0, 0)
    start(1, 1)

    def body(i, carry):
        for b in range(2):
            w = i * 2 + b
            wait(b)
            pltpu.sync_copy(buf.at[b], acc.at[ibuf.at[b]], add=True)

            @pl.when(w + 2 < NWIN)
            def _():
                start(w + 2, b)
        return carry

    lax.fori_loop(0, NWIN // 2, body, 0)

    pltpu.sync_copy(x_hbm.at[pl.ds(row0 + NWIN * W, TAIL)], tbx)
    pltpu.sync_copy(segs_hbm.at[pl.ds(row0 + NWIN * W, TAIL)], tbi.at[0])
    pltpu.sync_copy(tbx, acc.at[tbi.at[0]], add=True)

    plsc.subcore_barrier()

    @pl.when(cid == 0)
    def _():
        for j in range(625 // ZROWS):
            s = sid * 625 + j * ZROWS
            pltpu.sync_copy(acc.at[pl.ds(s, ZROWS)], y0_hbm.at[pl.ds(s, ZROWS)])

    @pl.when(cid == 1)
    def _():
        for j in range(625 // ZROWS):
            s = sid * 625 + j * ZROWS
            pltpu.sync_copy(acc.at[pl.ds(s, ZROWS)], y1_hbm.at[pl.ds(s, ZROWS)])


_pool_kernel = pl.kernel(
    _pool_body,
    out_type=(
        jax.ShapeDtypeStruct((NSEG, D), jnp.float32),
        jax.ShapeDtypeStruct((NSEG, D), jnp.float32),
    ),
    mesh=plsc.VectorSubcoreMesh(core_axis_name="c", subcore_axis_name="s"),
    scratch_types=[
        pltpu.VMEM_SHARED((NSEG, D), jnp.float32),
        pltpu.VMEM((2, W, D), jnp.float32),
        pltpu.VMEM((2, W), jnp.int32),
        pltpu.VMEM((ZROWS, D), jnp.float32),
        pltpu.VMEM((TAIL, D), jnp.float32),
        pltpu.VMEM((1, TAIL), jnp.int32),
        pltpu.SemaphoreType.DMA,
        pltpu.SemaphoreType.DMA,
        pltpu.SemaphoreType.DMA,
        pltpu.SemaphoreType.DMA,
    ],
)


def _gather_body(y0_hbm, y1_hbm, pool_hbm, out_hbm, pidx, r0, r1, s0, s1):
    cid = lax.axis_index("c")
    sid = lax.axis_index("s")
    g = cid * NS + sid

    for j in range((NGWIN + NW - 1) // NW):
        w = g + NW * j

        @pl.when(w < NGWIN)
        def _():
            base = w * GW
            pltpu.sync_copy(pool_hbm.at[pl.ds(base, GW)], pidx.at[0])
            c0 = pltpu.async_copy(y0_hbm.at[pidx.at[0]], r0, s0)
            c1 = pltpu.async_copy(y1_hbm.at[pidx.at[0]], r1, s1)
            c0.wait()
            c1.wait()

            def add_row(r, carry):
                for c in range(D // 16):
                    sl = pl.ds(c * 16, 16)
                    r0[r, sl] = r0[r, sl] + r1[r, sl]
                return carry

            lax.fori_loop(0, GW, add_row, 0)
            pltpu.sync_copy(r0, out_hbm.at[pl.ds(base, GW)])


_gather_kernel = pl.kernel(
    _gather_body,
    out_type=jax.ShapeDtypeStruct((NSEG, D), jnp.float32),
    mesh=plsc.VectorSubcoreMesh(core_axis_name="c", subcore_axis_name="s"),
    scratch_types=[
        pltpu.VMEM((1, GW), jnp.int32),
        pltpu.VMEM((GW, D), jnp.float32),
        pltpu.VMEM((GW, D), jnp.float32),
        pltpu.SemaphoreType.DMA,
        pltpu.SemaphoreType.DMA,
    ],
)


def kernel(x, segs, segs_pool):
    segs = segs.astype(jnp.int32)
    segs_pool = segs_pool.astype(jnp.int32)
    y0, y1 = _pool_kernel(x, segs)
    return _gather_kernel(y0, y1, segs_pool)


# speedup vs baseline: 7.0478x; 7.0478x over previous
import jax
import jax.numpy as jnp
from jax import lax
from jax.experimental import pallas as pl
from jax.experimental.pallas import tpu as pltpu
from jax.experimental.pallas import tpu_sc as plsc

N, D = 320000, 128
NSEG = 10000
NC, NS = 2, 16
NW = NC * NS
ROWS_PER_TILE = N // NW
W = 64
NWIN = ROWS_PER_TILE // W
TAIL = ROWS_PER_TILE - NWIN * W
NSEG_PAD = 10240
AROWS = NSEG_PAD // NS
ZROWS = 64
GW = 80
NGWIN = NSEG // GW


def _pool_body(x_hbm, segs_hbm, y0_hbm, y1_hbm,
               acc, buf, ibuf, zbuf, tbx, tbi, sx0, sx1, si0, si1):
    cid = lax.axis_index("c")
    sid = lax.axis_index("s")
    g = cid * NS + sid
    row0 = g * ROWS_PER_TILE

    z = jnp.zeros((16,), jnp.float32)

    def zrow(r, carry):
        for c in range(D // 16):
            zbuf[r, pl.ds(c * 16, 16)] = z
        return carry

    lax.fori_loop(0, ZROWS, zrow, 0)
    for j in range(AROWS // ZROWS):
        pltpu.sync_copy(zbuf, acc.at[pl.ds(sid * AROWS + j * ZROWS, ZROWS)])
    plsc.subcore_barrier()

    sx = (sx0, sx1)
    si = (si0, si1)

    def start(w, b):
        base = row0 + w * W
        pltpu.async_copy(x_hbm.at[pl.ds(base, W)], buf.at[b], sx[b])
        pltpu.async_copy(segs_hbm.at[pl.ds(base, W)], ibuf.at[b], si[b])

    def wait(b):
        pltpu.make_async_copy(x_hbm.at[pl.ds(0, W)], buf.at[b], sx[b]).wait()
        pltpu.make_async_copy(segs_hbm.at[pl.ds(0, W)], ibuf.at[b], si[b]).wait()

    start(0, 0)
    start(1, 1)

    def body(i, carry):
        for b in range(2):
            w = i * 2 + b
            wait(b)
            pltpu.sync_copy(buf.at[b], acc.at[ibuf.at[b]], add=True)

            @pl.when(w + 2 < NWIN)
            def _():
                start(w + 2, b)
        return carry

    lax.fori_loop(0, NWIN // 2, body, 0)

    pltpu.sync_copy(x_hbm.at[pl.ds(row0 + NWIN * W, TAIL)], tbx)
    pltpu.sync_copy(segs_hbm.at[pl.ds(row0 + NWIN * W, TAIL)], tbi.at[0])
    pltpu.sync_copy(tbx, acc.at[tbi.at[0]], add=True)

    plsc.subcore_barrier()

    @pl.when(cid == 0)
    def _():
        for j in range(AROWS // ZROWS):
            s = sid * AROWS + j * ZROWS
            pltpu.sync_copy(acc.at[pl.ds(s, ZROWS)], y0_hbm.at[pl.ds(s, ZROWS)])

    @pl.when(cid == 1)
    def _():
        for j in range(AROWS // ZROWS):
            s = sid * AROWS + j * ZROWS
            pltpu.sync_copy(acc.at[pl.ds(s, ZROWS)], y1_hbm.at[pl.ds(s, ZROWS)])


_pool_kernel = pl.kernel(
    _pool_body,
    out_type=(
        jax.ShapeDtypeStruct((NSEG_PAD, D), jnp.float32),
        jax.ShapeDtypeStruct((NSEG_PAD, D), jnp.float32),
    ),
    mesh=plsc.VectorSubcoreMesh(core_axis_name="c", subcore_axis_name="s"),
    scratch_types=[
        pltpu.VMEM_SHARED((NSEG_PAD, D), jnp.float32),
        pltpu.VMEM((2, W, D), jnp.float32),
        pltpu.VMEM((2, W), jnp.int32),
        pltpu.VMEM((ZROWS, D), jnp.float32),
        pltpu.VMEM((TAIL, D), jnp.float32),
        pltpu.VMEM((1, TAIL), jnp.int32),
        pltpu.SemaphoreType.DMA,
        pltpu.SemaphoreType.DMA,
        pltpu.SemaphoreType.DMA,
        pltpu.SemaphoreType.DMA,
    ],
)


def _gather_body(y0_hbm, y1_hbm, pool_hbm, out_hbm, pidx, r0, r1, s0, s1):
    cid = lax.axis_index("c")
    sid = lax.axis_index("s")
    g = cid * NS + sid

    for j in range((NGWIN + NW - 1) // NW):
        w = g + NW * j

        @pl.when(w < NGWIN)
        def _():
            base = w * GW
            pltpu.sync_copy(pool_hbm.at[pl.ds(base, GW)], pidx.at[0])
            c0 = pltpu.async_copy(y0_hbm.at[pidx.at[0]], r0, s0)
            c1 = pltpu.async_copy(y1_hbm.at[pidx.at[0]], r1, s1)
            c0.wait()
            c1.wait()

            def add_row(r, carry):
                for c in range(D // 16):
                    sl = pl.ds(c * 16, 16)
                    r0[r, sl] = r0[r, sl] + r1[r, sl]
                return carry

            lax.fori_loop(0, GW, add_row, 0)
            pltpu.sync_copy(r0, out_hbm.at[pl.ds(base, GW)])


_gather_kernel = pl.kernel(
    _gather_body,
    out_type=jax.ShapeDtypeStruct((NSEG, D), jnp.float32),
    mesh=plsc.VectorSubcoreMesh(core_axis_name="c", subcore_axis_name="s"),
    scratch_types=[
        pltpu.VMEM((1, GW), jnp.int32),
        pltpu.VMEM((GW, D), jnp.float32),
        pltpu.VMEM((GW, D), jnp.float32),
        pltpu.SemaphoreType.DMA,
        pltpu.SemaphoreType.DMA,
    ],
)


def kernel(x, segs, segs_pool):
    segs = segs.astype(jnp.int32)
    segs_pool = segs_pool.astype(jnp.int32)
    y0, y1 = _pool_kernel(x, segs)
    return _gather_kernel(y0, y1, segs_pool)
